# baseline (speedup 1.0000x reference)
import functools

import jax
import jax.numpy as jnp
from jax import lax
from jax.experimental import pallas as pl
from jax.experimental.pallas import tpu as pltpu
from jax.experimental.pallas import tpu_sc as plsc


def _make_sc_concat(b, n, d, p, plen, dtype):
    info = plsc.get_sparse_core_info()
    nw = info.num_cores * info.num_subcores
    assert n % nw == 0, (n, nw)
    rows = n // nw
    mesh = plsc.VectorSubcoreMesh(core_axis_name="c", subcore_axis_name="s")

    @functools.partial(
        pl.kernel,
        mesh=mesh,
        out_type=jax.ShapeDtypeStruct((b, plen + n, d), dtype),
        scratch_types=[pltpu.SemaphoreType.DMA, pltpu.SemaphoreType.DMA],
        compiler_params=pltpu.CompilerParams(use_tc_tiling_on_sc=False),
    )
    def sc_concat(feat_hbm, prompts_hbm, out_hbm, sem, psem):
        w = lax.axis_index("s") * info.num_cores + lax.axis_index("c")
        base = w * rows
        copies = []
        for bi in range(b):
            cp = pltpu.make_async_copy(
                feat_hbm.at[bi, pl.ds(base, rows), :],
                out_hbm.at[bi, pl.ds(plen + base, rows), :],
                sem)
            cp.start()
            copies.append(cp)

        @pl.when(w < b)
        def _():
            pcp = pltpu.make_async_copy(
                prompts_hbm.at[0],
                out_hbm.at[w, pl.ds(0, plen), :],
                psem)
            pcp.start()
            pcp.wait()

        for cp in copies:
            cp.wait()

    return sc_concat


def kernel(features, layer_idx, modality_indices, prompts, prompt_keys):
    del layer_idx, modality_indices
    del prompt_keys
    b, n, d = features.shape
    p, plen, _ = prompts.shape
    assert p == 1, "kernel exploits the single-prompt pool structure"
    sc_concat = _make_sc_concat(b, n, d, p, plen, features.dtype)
    return sc_concat(features, prompts)


# speedup vs baseline: 4.3646x; 4.3646x over previous
import functools

import jax
import jax.numpy as jnp
from jax import lax
from jax.experimental import pallas as pl
from jax.experimental.pallas import tpu as pltpu
from jax.experimental.pallas import tpu_sc as plsc

_CH = 32
_NBUF = 4


def _make_sc_concat(b, n, d, p, plen, dtype):
    info = plsc.get_sparse_core_info()
    nw = info.num_cores * info.num_subcores
    assert n % (nw * _CH) == 0, (n, nw)
    rows = n // nw
    nch = rows // _CH
    mesh = plsc.VectorSubcoreMesh(core_axis_name="c", subcore_axis_name="s")

    @functools.partial(
        pl.kernel,
        mesh=mesh,
        out_type=jax.ShapeDtypeStruct((b, plen + n, d), dtype),
        scratch_types=[
            pltpu.VMEM((_NBUF, _CH, d), dtype),
            pltpu.VMEM((plen, d), dtype),
            pltpu.SemaphoreType.DMA((_NBUF,)),
            pltpu.SemaphoreType.DMA((_NBUF,)),
            pltpu.SemaphoreType.DMA,
        ],
        compiler_params=pltpu.CompilerParams(use_tc_tiling_on_sc=False),
    )
    def sc_concat(feat_hbm, prompts_hbm, out_hbm, bufs, pbuf, isems, osems,
                  psem):
        w = lax.axis_index("s") * info.num_cores + lax.axis_index("c")
        base = w * rows

        def chunk_copies(i):
            bi, h = divmod(i, nch)
            r0 = base + h * _CH
            slot = i % _NBUF
            incp = pltpu.make_async_copy(
                feat_hbm.at[bi, pl.ds(r0, _CH), :],
                bufs.at[slot], isems.at[slot])
            outcp = pltpu.make_async_copy(
                bufs.at[slot],
                out_hbm.at[bi, pl.ds(plen + r0, _CH), :], osems.at[slot])
            return incp, outcp

        nchunks = b * nch
        cps = [chunk_copies(i) for i in range(nchunks)]
        for i in range(nchunks):
            if i >= _NBUF:
                cps[i - _NBUF][1].wait()
            cps[i][0].start()
            if i >= 1:
                cps[i - 1][0].wait()
                cps[i - 1][1].start()
        cps[-1][0].wait()
        cps[-1][1].start()
        for i in range(max(0, nchunks - _NBUF), nchunks):
            cps[i][1].wait()

        @pl.when(w < b)
        def _():
            pin = pltpu.make_async_copy(prompts_hbm.at[0], pbuf, psem)
            pin.start()
            pin.wait()
            pout = pltpu.make_async_copy(
                pbuf, out_hbm.at[w, pl.ds(0, plen), :], psem)
            pout.start()
            pout.wait()

    return sc_concat


def kernel(features, layer_idx, modality_indices, prompts, prompt_keys):
    del layer_idx, modality_indices
    del prompt_keys
    b, n, d = features.shape
    p, plen, _ = prompts.shape
    assert p == 1, "kernel exploits the single-prompt pool structure"
    sc_concat = _make_sc_concat(b, n, d, p, plen, features.dtype)
    return sc_concat(features, prompts)


# speedup vs baseline: 13.3449x; 3.0575x over previous
import functools

import jax
import jax.numpy as jnp
from jax.experimental import pallas as pl
from jax.experimental.pallas import tpu as pltpu

_BN = 512


def _body(nf, feat_ref, prompts_ref, keys_ref, out_ref, acc_ref, carry_ref,
          f0_ref):
    s = pl.program_id(1)
    bn = feat_ref.shape[1]
    n = nf * bn
    plen = prompts_ref.shape[1]
    p = prompts_ref.shape[0]

    @pl.when(s == 0)
    def _():
        acc_ref[...] = jnp.zeros_like(acc_ref)
        f0_ref[...] = feat_ref[0]

    @pl.when(s <= nf - 1)
    def _():
        f = feat_ref[0]
        acc_ref[...] += jnp.sum(f, axis=0, keepdims=True)
        rolled = pltpu.roll(f, plen, 0)

        @pl.when(s >= 1)
        def _():
            out_ref[0] = rolled
            out_ref[0, :plen, :] = carry_ref[...]

        carry_ref[...] = rolled[:plen]

    @pl.when(s == nf)
    def _():
        out_ref[0, :plen, :] = carry_ref[...]

    @pl.when(s == nf + 1)
    def _():
        q = acc_ref[...] * (1.0 / n)
        qn = q / jnp.maximum(jnp.sqrt(jnp.sum(q * q)), 1e-12)
        k = keys_ref[...]
        kn = k / jnp.maximum(
            jnp.sqrt(jnp.sum(k * k, axis=1, keepdims=True)), 1e-12)
        sim = jnp.sum(qn * kn, axis=1, keepdims=True)
        iota = jax.lax.broadcasted_iota(jnp.int32, sim.shape, 0)
        idx = jnp.min(jnp.where(sim >= jnp.max(sim), iota, p))
        out_ref[0] = pltpu.roll(f0_ref[...], plen, 0)
        out_ref[0, :plen, :] = prompts_ref[idx]


def kernel(features, layer_idx, modality_indices, prompts, prompt_keys):
    del layer_idx, modality_indices
    b, n, d = features.shape
    p, plen, _ = prompts.shape
    bn = _BN if n % _BN == 0 else n
    nf = n // bn

    def out_map(i, s):
        blk = jnp.where(s == nf + 1, 0, jnp.minimum(jnp.maximum(s, 1), nf))
        return (i, blk, 0)

    out = pl.pallas_call(
        functools.partial(_body, nf),
        grid=(b, nf + 2),
        in_specs=[
            pl.BlockSpec((1, bn, d),
                         lambda i, s: (i, jnp.minimum(s, nf - 1), 0)),
            pl.BlockSpec((p, plen, d), lambda i, s: (0, 0, 0)),
            pl.BlockSpec((p, d), lambda i, s: (0, 0)),
        ],
        out_specs=pl.BlockSpec((1, bn, d), out_map),
        out_shape=jax.ShapeDtypeStruct((b, plen + n, d), features.dtype),
        scratch_shapes=[
            pltpu.VMEM((1, d), jnp.float32),
            pltpu.VMEM((plen, d), jnp.float32),
            pltpu.VMEM((bn, d), jnp.float32),
        ],
    )(features, prompts, prompt_keys)
    return out
